# baseline (speedup 1.0000x reference)
import jax
import jax.numpy as jnp
import numpy as np
from jax.experimental import pallas as pl
from jax.experimental.pallas import tpu as pltpu

B = 128
V = 100000
BLK = 8192
NBLK = (V + BLK - 1) // BLK
RB = 128

_K0 = np.uint32(0)
_K1 = np.uint32(42)
_K2 = np.uint32(_K0 ^ _K1 ^ np.uint32(0x1BD11BDA))
_KS = (_K0, _K1, _K2)
_ROT0 = (13, 15, 26, 6)
_ROT1 = (17, 29, 16, 24)
_TINY = np.float32(np.finfo(np.float32).tiny)
_NEG_INF = np.float32(-np.inf)


def _rotl(x, d):
    return (x << np.uint32(d)) | (x >> np.uint32(32 - d))


def _threefry_bits(x1):
    x0 = jnp.full_like(x1, _KS[0])
    x1 = x1 + _KS[1]
    schedule = (
        (_ROT0, _KS[1], _KS[2], 1),
        (_ROT1, _KS[2], _KS[0], 2),
        (_ROT0, _KS[0], _KS[1], 3),
        (_ROT1, _KS[1], _KS[2], 4),
        (_ROT0, _KS[2], _KS[0], 5),
    )
    for rots, ka, kb, c in schedule:
        for d in rots:
            x0 = x0 + x1
            x1 = _rotl(x1, d)
            x1 = x1 ^ x0
        x0 = x0 + ka
        x1 = x1 + (kb + np.uint32(c))
    return x0 ^ x1


def _iota_u32(axis):
    return jax.lax.broadcasted_iota(jnp.uint32, (RB, BLK), axis)


def _scan_kernel(x_ref, a_ref, lp_ref, act_ref, m_ref, s_ref, zb_ref, lb_ref,
                 ab_ref):
    i = pl.program_id(0)
    j = pl.program_id(1)
    c0 = j * BLK

    col = _iota_u32(1) + c0.astype(jnp.uint32)
    row = _iota_u32(0) + (i * RB).astype(jnp.uint32)
    bits = _threefry_bits(row * np.uint32(V) + col)

    fl = jax.lax.bitcast_convert_type(
        (bits >> np.uint32(9)) | np.uint32(0x3F800000), jnp.float32
    ) - np.float32(1.0)
    g = -jnp.log(-jnp.log(fl + _TINY))

    x = x_ref[...]
    a_blk = a_ref[...]

    gcol = col.astype(jnp.int32)
    valid = gcol < V
    xm = jnp.where(valid, x, _NEG_INF)
    z = xm + g

    bmax = jnp.max(xm, axis=1, keepdims=True)
    bz = jnp.max(z, axis=1, keepdims=True)
    bidx = jnp.min(jnp.where(z == bz, gcol, V), axis=1, keepdims=True)
    at_best = gcol == bidx
    l_at = jnp.sum(jnp.where(at_best, x, np.float32(0.0)), axis=1,
                   keepdims=True)
    a_at = jnp.sum(jnp.where(at_best, a_blk, 0), axis=1, keepdims=True)

    @pl.when(j == 0)
    def _init():
        m_ref[...] = bmax
        s_ref[...] = jnp.sum(jnp.exp(xm - bmax), axis=1, keepdims=True)
        zb_ref[...] = bz
        lb_ref[...] = l_at
        ab_ref[...] = a_at

    @pl.when(j > 0)
    def _update():
        m_old = m_ref[...]
        m_new = jnp.maximum(m_old, bmax)
        bsum = jnp.sum(jnp.exp(xm - m_new), axis=1, keepdims=True)
        s_ref[...] = s_ref[...] * jnp.exp(m_old - m_new) + bsum
        m_ref[...] = m_new
        upd = bz > zb_ref[...]
        zb_ref[...] = jnp.where(upd, bz, zb_ref[...])
        lb_ref[...] = jnp.where(upd, l_at, lb_ref[...])
        ab_ref[...] = jnp.where(upd, a_at, ab_ref[...])

    @pl.when(j == NBLK - 1)
    def _fin():
        lp_ref[...] = lb_ref[...] - (m_ref[...] + jnp.log(s_ref[...]))
        act_ref[...] = ab_ref[...]


@jax.jit
def kernel(logits, action_space):
    a2d = action_space.reshape(1, V)
    lp, act = pl.pallas_call(
        _scan_kernel,
        grid=(B // RB, NBLK),
        in_specs=[
            pl.BlockSpec((RB, BLK), lambda i, j: (i, j)),
            pl.BlockSpec((1, BLK), lambda i, j: (0, j)),
        ],
        out_specs=[
            pl.BlockSpec((RB, 1), lambda i, j: (i, 0)),
            pl.BlockSpec((RB, 1), lambda i, j: (i, 0)),
        ],
        out_shape=[
            jax.ShapeDtypeStruct((B, 1), jnp.float32),
            jax.ShapeDtypeStruct((B, 1), jnp.int32),
        ],
        scratch_shapes=[
            pltpu.VMEM((RB, 1), jnp.float32),
            pltpu.VMEM((RB, 1), jnp.float32),
            pltpu.VMEM((RB, 1), jnp.float32),
            pltpu.VMEM((RB, 1), jnp.float32),
            pltpu.VMEM((RB, 1), jnp.int32),
        ],
        compiler_params=pltpu.CompilerParams(
            dimension_semantics=("parallel", "arbitrary"),
        ),
    )(logits, a2d)
    return lp.reshape(B), act.reshape(B)


# speedup vs baseline: 1.2759x; 1.2759x over previous
import jax
import jax.numpy as jnp
import numpy as np
from jax.experimental import pallas as pl
from jax.experimental.pallas import tpu as pltpu

B = 128
V = 100000
BLK = 1024
NBLK = (V + BLK - 1) // BLK
RB = 128

_K0 = np.uint32(0)
_K1 = np.uint32(42)
_K2 = np.uint32(_K0 ^ _K1 ^ np.uint32(0x1BD11BDA))
_KS = (_K0, _K1, _K2)
_ROT0 = (13, 15, 26, 6)
_ROT1 = (17, 29, 16, 24)
_TINY = np.float32(np.finfo(np.float32).tiny)
_NEG_INF = np.float32(-np.inf)


def _rotl(x, d):
    return (x << np.uint32(d)) | (x >> np.uint32(32 - d))


def _threefry_bits(x1):
    x0 = jnp.full_like(x1, _KS[0])
    x1 = x1 + _KS[1]
    schedule = (
        (_ROT0, _KS[1], _KS[2], 1),
        (_ROT1, _KS[2], _KS[0], 2),
        (_ROT0, _KS[0], _KS[1], 3),
        (_ROT1, _KS[1], _KS[2], 4),
        (_ROT0, _KS[2], _KS[0], 5),
    )
    for rots, ka, kb, c in schedule:
        for d in rots:
            x0 = x0 + x1
            x1 = _rotl(x1, d)
            x1 = x1 ^ x0
        x0 = x0 + ka
        x1 = x1 + (kb + np.uint32(c))
    return x0 ^ x1


def _iota_u32(axis):
    return jax.lax.broadcasted_iota(jnp.uint32, (RB, BLK), axis)


def _scan_kernel(x_ref, a_ref, lp_ref, act_ref, m_ref, s_ref, zb_ref, lb_ref,
                 ab_ref):
    i = pl.program_id(0)
    j = pl.program_id(1)
    c0 = j * BLK

    col = _iota_u32(1) + c0.astype(jnp.uint32)
    row = _iota_u32(0) + (i * RB).astype(jnp.uint32)
    bits = _threefry_bits(row * np.uint32(V) + col)

    fl = jax.lax.bitcast_convert_type(
        (bits >> np.uint32(9)) | np.uint32(0x3F800000), jnp.float32
    ) - np.float32(1.0)
    g = -jnp.log(-jnp.log(fl + _TINY))

    x = x_ref[...]
    a_blk = a_ref[...]

    gcol = col.astype(jnp.int32)
    valid = gcol < V
    xm = jnp.where(valid, x, _NEG_INF)
    z = xm + g

    bmax = jnp.max(xm, axis=1, keepdims=True)
    bz = jnp.max(z, axis=1, keepdims=True)
    bidx = jnp.min(jnp.where(z == bz, gcol, V), axis=1, keepdims=True)
    at_best = gcol == bidx
    l_at = jnp.sum(jnp.where(at_best, x, np.float32(0.0)), axis=1,
                   keepdims=True)
    a_at = jnp.sum(jnp.where(at_best, a_blk, 0), axis=1, keepdims=True)

    @pl.when(j == 0)
    def _init():
        m_ref[...] = bmax
        s_ref[...] = jnp.sum(jnp.exp(xm - bmax), axis=1, keepdims=True)
        zb_ref[...] = bz
        lb_ref[...] = l_at
        ab_ref[...] = a_at

    @pl.when(j > 0)
    def _update():
        m_old = m_ref[...]
        m_new = jnp.maximum(m_old, bmax)
        bsum = jnp.sum(jnp.exp(xm - m_new), axis=1, keepdims=True)
        s_ref[...] = s_ref[...] * jnp.exp(m_old - m_new) + bsum
        m_ref[...] = m_new
        upd = bz > zb_ref[...]
        zb_ref[...] = jnp.where(upd, bz, zb_ref[...])
        lb_ref[...] = jnp.where(upd, l_at, lb_ref[...])
        ab_ref[...] = jnp.where(upd, a_at, ab_ref[...])

    @pl.when(j == NBLK - 1)
    def _fin():
        lp_ref[...] = lb_ref[...] - (m_ref[...] + jnp.log(s_ref[...]))
        act_ref[...] = ab_ref[...]


@jax.jit
def kernel(logits, action_space):
    a2d = action_space.reshape(1, V)
    lp, act = pl.pallas_call(
        _scan_kernel,
        grid=(B // RB, NBLK),
        in_specs=[
            pl.BlockSpec((RB, BLK), lambda i, j: (i, j)),
            pl.BlockSpec((1, BLK), lambda i, j: (0, j)),
        ],
        out_specs=[
            pl.BlockSpec((RB, 1), lambda i, j: (i, 0)),
            pl.BlockSpec((RB, 1), lambda i, j: (i, 0)),
        ],
        out_shape=[
            jax.ShapeDtypeStruct((B, 1), jnp.float32),
            jax.ShapeDtypeStruct((B, 1), jnp.int32),
        ],
        scratch_shapes=[
            pltpu.VMEM((RB, 1), jnp.float32),
            pltpu.VMEM((RB, 1), jnp.float32),
            pltpu.VMEM((RB, 1), jnp.float32),
            pltpu.VMEM((RB, 1), jnp.float32),
            pltpu.VMEM((RB, 1), jnp.int32),
        ],
        compiler_params=pltpu.CompilerParams(
            dimension_semantics=("parallel", "arbitrary"),
        ),
    )(logits, a2d)
    return lp.reshape(B), act.reshape(B)


# speedup vs baseline: 1.3277x; 1.0406x over previous
import jax
import jax.numpy as jnp
import numpy as np
from jax.experimental import pallas as pl
from jax.experimental.pallas import tpu as pltpu

B = 128
V = 100000
BLK = 2048
NBLK = (V + BLK - 1) // BLK
RB = 128

_K0 = np.uint32(0)
_K1 = np.uint32(42)
_K2 = np.uint32(_K0 ^ _K1 ^ np.uint32(0x1BD11BDA))
_KS = (_K0, _K1, _K2)
_ROT0 = (13, 15, 26, 6)
_ROT1 = (17, 29, 16, 24)
_TINY = np.float32(np.finfo(np.float32).tiny)
_NEG_INF = np.float32(-np.inf)


def _rotl(x, d):
    return (x << np.uint32(d)) | (x >> np.uint32(32 - d))


def _threefry_bits(x1):
    x0 = jnp.full_like(x1, _KS[0])
    x1 = x1 + _KS[1]
    schedule = (
        (_ROT0, _KS[1], _KS[2], 1),
        (_ROT1, _KS[2], _KS[0], 2),
        (_ROT0, _KS[0], _KS[1], 3),
        (_ROT1, _KS[1], _KS[2], 4),
        (_ROT0, _KS[2], _KS[0], 5),
    )
    for rots, ka, kb, c in schedule:
        for d in rots:
            x0 = x0 + x1
            x1 = _rotl(x1, d)
            x1 = x1 ^ x0
        x0 = x0 + ka
        x1 = x1 + (kb + np.uint32(c))
    return x0 ^ x1


def _iota_u32(axis):
    return jax.lax.broadcasted_iota(jnp.uint32, (RB, BLK), axis)


def _scan_kernel(x_ref, a_ref, lp_ref, act_ref, m_ref, s_ref, zb_ref, lb_ref,
                 ab_ref):
    i = pl.program_id(0)
    j = pl.program_id(1)
    c0 = j * BLK

    col = _iota_u32(1) + c0.astype(jnp.uint32)
    row = _iota_u32(0) + (i * RB).astype(jnp.uint32)
    bits = _threefry_bits(row * np.uint32(V) + col)

    fl = jax.lax.bitcast_convert_type(
        (bits >> np.uint32(9)) | np.uint32(0x3F800000), jnp.float32
    ) - np.float32(1.0)
    g = -jnp.log(-jnp.log(fl + _TINY))

    x = x_ref[...]
    a_blk = a_ref[...]

    gcol = col.astype(jnp.int32)
    valid = gcol < V
    xm = jnp.where(valid, x, _NEG_INF)
    z = xm + g

    bmax = jnp.max(xm, axis=1, keepdims=True)
    bz = jnp.max(z, axis=1, keepdims=True)
    bidx = jnp.min(jnp.where(z == bz, gcol, V), axis=1, keepdims=True)
    at_best = gcol == bidx
    l_at = jnp.sum(jnp.where(at_best, x, np.float32(0.0)), axis=1,
                   keepdims=True)
    a_at = jnp.sum(jnp.where(at_best, a_blk, 0), axis=1, keepdims=True)

    @pl.when(j == 0)
    def _init():
        m_ref[...] = bmax
        s_ref[...] = jnp.sum(jnp.exp(xm - bmax), axis=1, keepdims=True)
        zb_ref[...] = bz
        lb_ref[...] = l_at
        ab_ref[...] = a_at

    @pl.when(j > 0)
    def _update():
        m_old = m_ref[...]
        m_new = jnp.maximum(m_old, bmax)
        bsum = jnp.sum(jnp.exp(xm - m_new), axis=1, keepdims=True)
        s_ref[...] = s_ref[...] * jnp.exp(m_old - m_new) + bsum
        m_ref[...] = m_new
        upd = bz > zb_ref[...]
        zb_ref[...] = jnp.where(upd, bz, zb_ref[...])
        lb_ref[...] = jnp.where(upd, l_at, lb_ref[...])
        ab_ref[...] = jnp.where(upd, a_at, ab_ref[...])

    @pl.when(j == NBLK - 1)
    def _fin():
        lp_ref[...] = lb_ref[...] - (m_ref[...] + jnp.log(s_ref[...]))
        act_ref[...] = ab_ref[...]


@jax.jit
def kernel(logits, action_space):
    a2d = action_space.reshape(1, V)
    lp, act = pl.pallas_call(
        _scan_kernel,
        grid=(B // RB, NBLK),
        in_specs=[
            pl.BlockSpec((RB, BLK), lambda i, j: (i, j)),
            pl.BlockSpec((1, BLK), lambda i, j: (0, j)),
        ],
        out_specs=[
            pl.BlockSpec((RB, 1), lambda i, j: (i, 0)),
            pl.BlockSpec((RB, 1), lambda i, j: (i, 0)),
        ],
        out_shape=[
            jax.ShapeDtypeStruct((B, 1), jnp.float32),
            jax.ShapeDtypeStruct((B, 1), jnp.int32),
        ],
        scratch_shapes=[
            pltpu.VMEM((RB, 1), jnp.float32),
            pltpu.VMEM((RB, 1), jnp.float32),
            pltpu.VMEM((RB, 1), jnp.float32),
            pltpu.VMEM((RB, 1), jnp.float32),
            pltpu.VMEM((RB, 1), jnp.int32),
        ],
        compiler_params=pltpu.CompilerParams(
            dimension_semantics=("parallel", "arbitrary"),
        ),
    )(logits, a2d)
    return lp.reshape(B), act.reshape(B)
